# baseline (speedup 1.0000x reference)
import functools

import jax
import jax.numpy as jnp
from jax import lax
from jax.experimental import pallas as pl
from jax.experimental.pallas import tpu as pltpu
from jax.experimental.pallas import tpu_sc as plsc

_NC = 2
_NS = 16
_NW = _NC * _NS
_LANES = 128


def _sc_coeff_kernel(rows_w, c_size, n_c):
    c_slice = c_size // _NS

    mesh = plsc.VectorSubcoreMesh(core_axis_name="c", subcore_axis_name="s")

    @functools.partial(
        pl.kernel,
        mesh=mesh,
        out_type=jax.ShapeDtypeStruct((_NC * c_size,), jnp.float32),
        scratch_types=[
            pltpu.VMEM((rows_w, _LANES), jnp.int32),
            pltpu.VMEM((rows_w, _LANES), jnp.int32),
            pltpu.VMEM((rows_w, _LANES), jnp.float32),
            pltpu.VMEM((c_size // _NS,), jnp.float32),
            pltpu.VMEM_SHARED((c_size,), jnp.float32),
            pltpu.SemaphoreType.DMA,
        ],
    )
    def sc_kernel(src_hbm, typ_hbm, norm_hbm, out_hbm,
                  src_v, idx_v, upd_v, zbuf_v, acc_sh, sem):
        cid = lax.axis_index("c")
        sid = lax.axis_index("s")
        wid = sid * _NC + cid

        base = wid * rows_w
        zbase = sid * c_slice
        d_src = pltpu.async_copy(src_hbm.at[pl.ds(base, rows_w)], src_v, sem)
        d_typ = pltpu.async_copy(typ_hbm.at[pl.ds(base, rows_w)], idx_v, sem)
        d_nrm = pltpu.async_copy(norm_hbm.at[pl.ds(base, rows_w)], upd_v, sem)

        def _zrow(i, carry):
            zbuf_v[pl.ds(i * 16, 16)] = jnp.zeros((16,), jnp.float32)
            return carry

        lax.fori_loop(0, c_slice // 16, _zrow, 0)
        d_zero = pltpu.async_copy(zbuf_v, acc_sh.at[pl.ds(zbase, c_slice)], sem)
        d_src.wait()
        d_typ.wait()

        def _row(i, carry):
            for j in range(_LANES // 16):
                sl = pl.ds(j * 16, 16)
                idx_v[i, sl] = idx_v[i, sl] * n_c + src_v[i, sl]
            return carry

        lax.fori_loop(0, rows_w, _row, 0)
        d_nrm.wait()
        d_zero.wait()

        plsc.subcore_barrier()

        def _fire(i, carry):
            pltpu.async_copy(upd_v.at[i], acc_sh.at[idx_v.at[i]], sem,
                             add=True)
            return carry

        lax.fori_loop(0, rows_w, _fire, 0)

        def _drain(i, carry):
            pltpu.make_async_copy(upd_v.at[i], acc_sh.at[idx_v.at[i]],
                                  sem).wait()
            return carry

        lax.fori_loop(0, rows_w, _drain, 0)
        plsc.subcore_barrier()

        pltpu.sync_copy(acc_sh.at[pl.ds(zbase, c_slice)],
                        out_hbm.at[pl.ds(cid * c_size + zbase, c_slice)])

    return sc_kernel


def _tc_root_body(x_ref, w_ref, o_ref):
    o_ref[...] = jnp.dot(x_ref[...], w_ref[...],
                         preferred_element_type=jnp.float32
                         ).astype(jnp.bfloat16)


def _tc_body(x_ref, w_ref, root_ref, *refs):
    c_refs, o_ref = refs[:-1], refs[-1]
    nrel = w_ref.shape[0]
    xb = x_ref[...]
    parts = [root_ref[...].astype(jnp.float32)]
    for r in range(nrel):
        c = c_refs[r][...] + c_refs[nrel + r][...]
        cb = c.astype(jnp.bfloat16)
        parts.append(jnp.dot(xb * cb[:, None], w_ref[r],
                             preferred_element_type=jnp.float32))
    while len(parts) > 1:
        parts = [a + b for a, b in zip(parts[::2], parts[1::2])] + (
            [parts[-1]] if len(parts) % 2 else [])
    o_ref[...] = parts[0]


def kernel(x, edge_index, edge_type, edge_norm, dim, W, root):
    n, d = x.shape
    r = W.shape[0]
    o = root.shape[1]
    e = edge_type.shape[0]

    blk = 2048
    grid = -(-n // blk)
    rows_w = -(-e // (_NW * _LANES))
    e_pad = _NW * rows_w * _LANES
    n_c = -(-n // 256) * 256
    c_size = r * n_c

    src_p = jnp.pad(edge_index[0], (0, e_pad - e)).reshape(_NW * rows_w, _LANES)
    typ_p = jnp.pad(edge_type, (0, e_pad - e)).reshape(_NW * rows_w, _LANES)
    norm_p = jnp.pad(edge_norm, (0, e_pad - e)).reshape(_NW * rows_w, _LANES)

    c_flat = _sc_coeff_kernel(rows_w, c_size, n_c)(src_p, typ_p, norm_p)

    w_b = W.astype(jnp.bfloat16)
    root_b = root.astype(jnp.bfloat16)
    x_b = x.astype(jnp.bfloat16)

    out_root = pl.pallas_call(
        _tc_root_body,
        grid=(grid,),
        in_specs=[
            pl.BlockSpec((blk, d), lambda i: (i, 0)),
            pl.BlockSpec((d, o), lambda i: (0, 0)),
        ],
        out_specs=pl.BlockSpec((blk, o), lambda i: (i, 0)),
        out_shape=jax.ShapeDtypeStruct((n, o), jnp.bfloat16),
    )(x_b, root_b)

    npb = n_c // blk
    c_specs = [
        pl.BlockSpec((blk,), functools.partial(
            lambda p_, r_, i: (p_ * r * npb + r_ * npb + i,), p_, r_))
        for p_ in range(_NC) for r_ in range(r)
    ]

    out = pl.pallas_call(
        _tc_body,
        grid=(grid,),
        in_specs=[
            pl.BlockSpec((blk, d), lambda i: (i, 0)),
            pl.BlockSpec((r, d, o), lambda i: (0, 0, 0)),
            pl.BlockSpec((blk, o), lambda i: (i, 0)),
        ] + c_specs,
        out_specs=pl.BlockSpec((blk, o), lambda i: (i, 0)),
        out_shape=jax.ShapeDtypeStruct((n, o), jnp.float32),
    )(x_b, w_b, out_root, *([c_flat] * (_NC * r)))
    return out


# speedup vs baseline: 1.1606x; 1.1606x over previous
import functools

import jax
import jax.numpy as jnp
from jax import lax
from jax.experimental import pallas as pl
from jax.experimental.pallas import tpu as pltpu
from jax.experimental.pallas import tpu_sc as plsc

_NC = 2
_NS = 16
_NW = _NC * _NS
_LANES = 128


def _sc_coeff_kernel(rows_w, c_size, n_c):
    c_slice = c_size // _NS

    mesh = plsc.VectorSubcoreMesh(core_axis_name="c", subcore_axis_name="s")

    @functools.partial(
        pl.kernel,
        mesh=mesh,
        out_type=jax.ShapeDtypeStruct((_NC * c_size,), jnp.float32),
        scratch_types=[
            pltpu.VMEM((rows_w, _LANES), jnp.int32),
            pltpu.VMEM((rows_w, _LANES), jnp.int32),
            pltpu.VMEM((rows_w, _LANES), jnp.float32),
            pltpu.VMEM((c_size // _NS,), jnp.float32),
            pltpu.VMEM_SHARED((c_size,), jnp.float32),
            pltpu.SemaphoreType.DMA,
        ],
    )
    def sc_kernel(ei_hbm, typ_hbm, norm_hbm, out_hbm,
                  src_v, idx_v, upd_v, zbuf_v, acc_sh, sem):
        cid = lax.axis_index("c")
        sid = lax.axis_index("s")
        wid = sid * _NC + cid

        base = wid * rows_w
        zbase = sid * c_slice
        d_src = pltpu.async_copy(ei_hbm.at[0, pl.ds(base, rows_w)], src_v, sem)
        d_typ = pltpu.async_copy(typ_hbm.at[pl.ds(base, rows_w)], idx_v, sem)
        d_nrm = pltpu.async_copy(norm_hbm.at[pl.ds(base, rows_w)], upd_v, sem)

        def _zrow(i, carry):
            zbuf_v[pl.ds(i * 16, 16)] = jnp.zeros((16,), jnp.float32)
            return carry

        lax.fori_loop(0, c_slice // 16, _zrow, 0)
        d_zero = pltpu.async_copy(zbuf_v, acc_sh.at[pl.ds(zbase, c_slice)], sem)
        d_src.wait()
        d_typ.wait()

        def _row(i, carry):
            for j in range(_LANES // 16):
                sl = pl.ds(j * 16, 16)
                idx_v[i, sl] = idx_v[i, sl] * n_c + src_v[i, sl]
            return carry

        lax.fori_loop(0, rows_w, _row, 0)
        d_nrm.wait()
        d_zero.wait()

        plsc.subcore_barrier()

        def _fire(i, carry):
            pltpu.async_copy(upd_v.at[i], acc_sh.at[idx_v.at[i]], sem,
                             add=True)
            return carry

        lax.fori_loop(0, rows_w, _fire, 0)

        def _drain(i, carry):
            pltpu.make_async_copy(upd_v.at[i], acc_sh.at[idx_v.at[i]],
                                  sem).wait()
            return carry

        lax.fori_loop(0, rows_w, _drain, 0)
        plsc.subcore_barrier()

        pltpu.sync_copy(acc_sh.at[pl.ds(zbase, c_slice)],
                        out_hbm.at[pl.ds(cid * c_size + zbase, c_slice)])

    return sc_kernel


def _tc_root_body(x_ref, w_ref, o_ref):
    o_ref[...] = jnp.dot(x_ref[...], w_ref[...],
                         preferred_element_type=jnp.float32
                         ).astype(jnp.bfloat16)


def _tc_body(x_ref, w_ref, root_ref, *refs):
    c_refs, o_ref = refs[:-1], refs[-1]
    nrel = w_ref.shape[0]
    xb = x_ref[...]
    parts = [root_ref[...].astype(jnp.float32)]
    for r in range(nrel):
        c = c_refs[r][...] + c_refs[nrel + r][...]
        cb = c.astype(jnp.bfloat16)
        parts.append(jnp.dot(xb * cb[:, None], w_ref[r],
                             preferred_element_type=jnp.float32))
    while len(parts) > 1:
        parts = [a + b for a, b in zip(parts[::2], parts[1::2])] + (
            [parts[-1]] if len(parts) % 2 else [])
    o_ref[...] = parts[0]


def kernel(x, edge_index, edge_type, edge_norm, dim, W, root):
    n, d = x.shape
    r = W.shape[0]
    o = root.shape[1]
    e = edge_type.shape[0]

    blk = 2048
    grid = -(-n // blk)
    rows_w = -(-e // (_NW * _LANES))
    e_pad = _NW * rows_w * _LANES
    n_c = -(-n // 256) * 256
    c_size = r * n_c

    ei_p = jnp.pad(edge_index, ((0, 0), (0, e_pad - e)))
    ei_p = ei_p.reshape(2, _NW * rows_w, _LANES)
    typ_p = jnp.pad(edge_type, (0, e_pad - e)).reshape(_NW * rows_w, _LANES)
    norm_p = jnp.pad(edge_norm, (0, e_pad - e)).reshape(_NW * rows_w, _LANES)

    c_flat = _sc_coeff_kernel(rows_w, c_size, n_c)(ei_p, typ_p, norm_p)

    w_b = W.astype(jnp.bfloat16)
    root_b = root.astype(jnp.bfloat16)
    x_b = x.astype(jnp.bfloat16)

    out_root = pl.pallas_call(
        _tc_root_body,
        grid=(grid,),
        in_specs=[
            pl.BlockSpec((blk, d), lambda i: (i, 0)),
            pl.BlockSpec((d, o), lambda i: (0, 0)),
        ],
        out_specs=pl.BlockSpec((blk, o), lambda i: (i, 0)),
        out_shape=jax.ShapeDtypeStruct((n, o), jnp.bfloat16),
    )(x_b, root_b)

    npb = n_c // blk
    c_specs = [
        pl.BlockSpec((blk,), functools.partial(
            lambda p_, r_, i: (p_ * r * npb + r_ * npb + i,), p_, r_))
        for p_ in range(_NC) for r_ in range(r)
    ]

    out = pl.pallas_call(
        _tc_body,
        grid=(grid,),
        in_specs=[
            pl.BlockSpec((blk, d), lambda i: (i, 0)),
            pl.BlockSpec((r, d, o), lambda i: (0, 0, 0)),
            pl.BlockSpec((blk, o), lambda i: (i, 0)),
        ] + c_specs,
        out_specs=pl.BlockSpec((blk, o), lambda i: (i, 0)),
        out_shape=jax.ShapeDtypeStruct((n, o), jnp.float32),
    )(x_b, w_b, out_root, *([c_flat] * (_NC * r)))
    return out
